# baseline (speedup 1.0000x reference)
import functools

import jax
import jax.numpy as jnp
from jax import lax
from jax.experimental import pallas as pl
from jax.experimental.pallas import tpu as pltpu

_N = 10000
_K = 9
_NP = 10240
_BR = 64
_PAD_COORD = 1000.0
_REMOVED = 0x7F000000
_IDX_MASK = 16383


def _fused_body(n, k, npad, br, vfull_ref, vrow_ref, out_ref, s1, s2):
    nb = npad // br
    step = pl.program_id(0)

    @pl.when(step == 0)
    def _init():
        s2[...] = jnp.zeros_like(s2)

    @pl.when(step < nb)
    def _main():
        vp = vfull_ref[...]
        vr = vrow_ref[...]
        sq_all = jnp.sum(vp * vp, axis=1)
        sq_r = jnp.sum(vr * vr, axis=1)
        dot = lax.dot_general(
            vr, vp, (((1,), (1,)), ((), ())),
            preferred_element_type=jnp.float32,
            precision=lax.Precision.HIGHEST)
        d2 = jnp.maximum(sq_r[:, None] + sq_all[None, :] - 2.0 * dot, 0.0)
        cols = lax.broadcasted_iota(jnp.int32, (br, npad), 1)
        key = (lax.bitcast_convert_type(d2, jnp.int32) & ~_IDX_MASK) | cols

        a = jnp.zeros((br, npad), jnp.float32)
        for it in range(k + 1):
            m = jnp.min(key, axis=1)
            sel = key == m[:, None]
            if it > 0:
                a += sel.astype(jnp.float32)
            key = jnp.where(sel, jnp.int32(_REMOVED), key)
        row0 = step * br
        rvalid = (row0 + lax.broadcasted_iota(jnp.int32, (br, 1), 0)) < n
        a = jnp.where(rvalid, a, 0.0)

        one3 = (lax.broadcasted_iota(jnp.int32, (npad, 4), 1) == 3)
        v4 = jnp.where(one3, 1.0, vp)
        one3r = (lax.broadcasted_iota(jnp.int32, (br, 4), 1) == 3)
        v4r = jnp.where(one3r, 1.0, vr)
        s1[pl.ds(row0, br), :] = lax.dot_general(
            a, v4, (((1,), (0,)), ((), ())),
            preferred_element_type=jnp.float32,
            precision=lax.Precision.HIGHEST)
        s2[...] += lax.dot_general(
            a, v4r, (((0,), (0,)), ((), ())),
            preferred_element_type=jnp.float32,
            precision=lax.Precision.HIGHEST)

    @pl.when(step == nb)
    def _fin():
        t = s1[...] + s2[...]
        vp = vfull_ref[...]
        deg = t[:, 3:4]
        lv = t[:, 0:3] / deg - vp[:, 0:3]
        nrm = jnp.sqrt(jnp.sum(lv * lv, axis=1, keepdims=True))
        valid = lax.broadcasted_iota(jnp.int32, (npad, 1), 0) < n
        loss = jnp.sum(jnp.where(valid, nrm, 0.0)) / n
        out_ref[...] = loss[None, None]


def _fused_call(n, k, npad, br, vp, interpret=False):
    nb = npad // br
    return pl.pallas_call(
        functools.partial(_fused_body, n, k, npad, br),
        grid=(nb + 1,),
        in_specs=[
            pl.BlockSpec((npad, 4), lambda i: (0, 0)),
            pl.BlockSpec((br, 4), lambda i: (jnp.minimum(i, nb - 1), 0)),
        ],
        out_specs=pl.BlockSpec((1, 1), lambda i: (0, 0)),
        out_shape=jax.ShapeDtypeStruct((1, 1), jnp.float32),
        scratch_shapes=[
            pltpu.VMEM((npad, 4), jnp.float32),
            pltpu.VMEM((npad, 4), jnp.float32),
        ],
        interpret=interpret,
    )(vp, vp)


def kernel(xyz_canon):
    vp = jnp.full((_NP, 4), _PAD_COORD, jnp.float32)
    vp = vp.at[:, 3].set(0.0).at[:_N, :3].set(xyz_canon)
    loss = _fused_call(_N, _K, _NP, _BR, vp)
    return loss[0, 0]


# speedup vs baseline: 2.9648x; 2.9648x over previous
import dataclasses
import functools

import jax
import jax.numpy as jnp
from jax import lax
from jax.experimental import pallas as pl
from jax.experimental.pallas import tpu as pltpu
from jax.experimental.pallas import tpu_sc as plsc

_N = 10000
_K = 9
_NP = 10240
_BR = 128
_PAD_COORD = 1000.0
_REMOVED = 0x7F000000
_IDX_MASK = 16383
_NTILES = 32
_RPT = _NP // _NTILES
_L = 16


def _knn_body(n, k, npad, br, vfull_ref, vrow_ref, out_ref):
    step = pl.program_id(0)
    vp = vfull_ref[...]
    vr = vrow_ref[...]
    sq_all = jnp.sum(vp * vp, axis=1)
    sq_r = jnp.sum(vr * vr, axis=1)
    dot = lax.dot_general(
        vr, vp, (((1,), (1,)), ((), ())),
        preferred_element_type=jnp.float32,
        precision=lax.Precision.HIGHEST)
    d2 = jnp.maximum(sq_r[:, None] + sq_all[None, :] - 2.0 * dot, 0.0)
    cols = lax.broadcasted_iota(jnp.int32, (br, npad), 1)
    key = (lax.bitcast_convert_type(d2, jnp.int32) & ~_IDX_MASK) | cols

    row0 = step * br
    lvalid = (row0 + lax.iota(jnp.int32, br)) < n
    trash = jnp.full((br,), npad - 1, jnp.int32)
    for it in range(k + 1):
        m = jnp.min(key, axis=1)
        idx = jnp.where(lvalid, m & _IDX_MASK, trash)
        out_ref[it, :] = idx
        key = jnp.where(key == m[:, None], jnp.int32(_REMOVED), key)


def _knn_call(n, k, npad, br, vp):
    nb = npad // br
    return pl.pallas_call(
        functools.partial(_knn_body, n, k, npad, br),
        grid=(nb,),
        in_specs=[
            pl.BlockSpec((npad, 4), lambda i: (0, 0)),
            pl.BlockSpec((br, 4), lambda i: (i, 0)),
        ],
        out_specs=pl.BlockSpec((16, br), lambda i: (0, i)),
        out_shape=jax.ShapeDtypeStruct((16, npad), jnp.int32),
    )(vp, vp)


def _sc_scatter(npad, k, knn9r, knn_rm, xyzt3):
    mesh = plsc.VectorSubcoreMesh(core_axis_name="c", subcore_axis_name="s")
    rpt = npad // _NTILES
    cp = pltpu.CompilerParams()
    if "needs_layout_passes" in pltpu.CompilerParams.__dataclass_fields__:
        cp = dataclasses.replace(cp, needs_layout_passes=False)

    @functools.partial(
        pl.kernel,
        mesh=mesh,
        compiler_params=cp,
        out_type=jax.ShapeDtypeStruct((_NTILES, 4, 1, npad), jnp.float32),
        scratch_types=[
            pltpu.VMEM((npad,), jnp.float32),
            pltpu.VMEM((npad,), jnp.float32),
            pltpu.VMEM((npad,), jnp.float32),
            pltpu.VMEM((npad,), jnp.float32),
            pltpu.VMEM((npad,), jnp.float32),
            pltpu.VMEM((npad,), jnp.float32),
            pltpu.VMEM((npad,), jnp.float32),
            pltpu.VMEM((k, rpt), jnp.int32),
            pltpu.VMEM((rpt, 16), jnp.int32),
            pltpu.SemaphoreType.DMA,
        ],
    )
    def sck(knn9r_hbm, knnrm_hbm, xyzt_hbm, out_hbm,
            xv, yv, zv, sx, sy, sz, dg, ktb, krb, sem):
        wid = lax.axis_index("c") * 16 + lax.axis_index("s")
        base = wid * rpt

        zero16 = jnp.zeros((_L,), jnp.float32)

        @pl.loop(0, npad // _L)
        def _(i):
            sl = pl.ds(i * _L, _L)
            sx[sl] = zero16
            sy[sl] = zero16
            sz[sl] = zero16
            dg[sl] = zero16

        pltpu.sync_copy(xyzt_hbm.at[0, 0], xv)
        pltpu.sync_copy(xyzt_hbm.at[1, 0], yv)
        pltpu.sync_copy(xyzt_hbm.at[2, 0], zv)
        pltpu.sync_copy(knn9r_hbm.at[wid], ktb)
        pltpu.sync_copy(knnrm_hbm.at[pl.ds(base, rpt)], krb)

        @pl.loop(0, rpt // _L)
        def _(g):
            sl = pl.ds(g * _L, _L)
            accx = jnp.zeros((_L,), jnp.float32)
            accy = jnp.zeros((_L,), jnp.float32)
            accz = jnp.zeros((_L,), jnp.float32)
            for kk in range(k):
                u = ktb[kk, sl]
                accx += plsc.load_gather(xv, [u])
                accy += plsc.load_gather(yv, [u])
                accz += plsc.load_gather(zv, [u])
            gsl = pl.ds(base + g * _L, _L)
            sx[gsl] = sx[gsl] + accx
            sy[gsl] = sy[gsl] + accy
            sz[gsl] = sz[gsl] + accz

        ones16 = jnp.ones((_L,), jnp.float32)
        mask9 = lax.iota(jnp.int32, _L) < k
        zero16i = jnp.zeros((_L,), jnp.int32)

        @pl.loop(0, rpt)
        def _(r):
            u = krb[r]
            rg = zero16i + (base + r)
            vx = plsc.load_gather(xv, [rg])
            vy = plsc.load_gather(yv, [rg])
            vz = plsc.load_gather(zv, [rg])
            plsc.addupdate_scatter(sx, [u], vx, mask=mask9)
            plsc.addupdate_scatter(sy, [u], vy, mask=mask9)
            plsc.addupdate_scatter(sz, [u], vz, mask=mask9)
            plsc.addupdate_scatter(dg, [u], ones16, mask=mask9)

        pltpu.sync_copy(sx, out_hbm.at[wid, 0, 0])
        pltpu.sync_copy(sy, out_hbm.at[wid, 1, 0])
        pltpu.sync_copy(sz, out_hbm.at[wid, 2, 0])
        pltpu.sync_copy(dg, out_hbm.at[wid, 3, 0])

    return sck(knn9r, knn_rm, xyzt3)


def _finish_body(n, k, npad, parts_ref, xyzt_ref, out_ref):
    acc = parts_ref[0]
    for t in range(1, _NTILES):
        acc = acc + parts_ref[t]
    deg = acc[3:4, :] + jnp.float32(k)
    lx = acc[0:1, :] / deg - xyzt_ref[0:1, :]
    ly = acc[1:2, :] / deg - xyzt_ref[1:2, :]
    lz = acc[2:3, :] / deg - xyzt_ref[2:3, :]
    nrm = jnp.sqrt(lx * lx + ly * ly + lz * lz)
    valid = lax.broadcasted_iota(jnp.int32, (1, npad), 1) < n
    loss = jnp.sum(jnp.where(valid, nrm, 0.0)) / n
    out_ref[...] = loss[None, None]


def _finish_call(n, k, npad, parts, xyzt):
    return pl.pallas_call(
        functools.partial(_finish_body, n, k, npad),
        out_shape=jax.ShapeDtypeStruct((1, 1), jnp.float32),
    )(parts, xyzt)


def kernel(xyz_canon):
    vp = jnp.full((_NP, 4), _PAD_COORD, jnp.float32)
    vp = vp.at[:, 3].set(0.0).at[:_N, :3].set(xyz_canon)
    xyzt = jnp.full((3, _NP), _PAD_COORD, jnp.float32)
    xyzt = xyzt.at[:, :_N].set(xyz_canon.T)

    knnt = _knn_call(_N, _K, _NP, _BR, vp)
    knn9 = knnt[1:1 + _K]
    knn9r = knn9.reshape(_K, _NTILES, _RPT).transpose(1, 0, 2)
    knn_rm = jnp.pad(knn9.T, ((0, 0), (0, 16 - _K)))
    parts = _sc_scatter(_NP, _K, knn9r, knn_rm, xyzt.reshape(3, 1, _NP))
    loss = _finish_call(_N, _K, _NP, parts.reshape(_NTILES, 4, _NP), xyzt)
    return loss[0, 0]


# speedup vs baseline: 3.0138x; 1.0166x over previous
import dataclasses
import functools

import jax
import jax.numpy as jnp
from jax import lax
from jax.experimental import pallas as pl
from jax.experimental.pallas import tpu as pltpu
from jax.experimental.pallas import tpu_sc as plsc

_N = 10000
_K = 9
_NP = 10240
_BR = 128
_PAD_COORD = 1000.0
_REMOVED = 0x7F000000
_IDX_MASK = 16383
_NTILES = 32
_RPT = _NP // _NTILES
_L = 16


def _knn_body(n, k, npad, br, vfull_ref, vrow_ref, out_ref):
    step = pl.program_id(0)
    vp = vfull_ref[...]
    vr = vrow_ref[...]
    sq_all = jnp.sum(vp * vp, axis=1)
    sq_r = jnp.sum(vr * vr, axis=1)
    w = jnp.concatenate(
        [-2.0 * vp[:, 0:3], jnp.ones((npad, 1), jnp.float32),
         sq_all[:, None]], axis=1)
    u = jnp.concatenate(
        [vr[:, 0:3], sq_r[:, None],
         jnp.ones((br, 1), jnp.float32)], axis=1)
    d2 = lax.dot_general(
        u, w, (((1,), (1,)), ((), ())),
        preferred_element_type=jnp.float32,
        precision=lax.Precision.HIGHEST)
    cols = lax.broadcasted_iota(jnp.int32, (br, npad), 1)
    key = (lax.bitcast_convert_type(d2, jnp.int32) & ~_IDX_MASK) | cols

    row0 = step * br
    lvalid = (row0 + lax.iota(jnp.int32, br)) < n
    trash = jnp.full((br,), npad - 1, jnp.int32)
    m = jnp.min(key, axis=1)
    for it in range(1, k + 1):
        m = jnp.min(jnp.where(key > m[:, None], key, jnp.int32(_REMOVED)),
                    axis=1)
        idx = jnp.where(lvalid, m & _IDX_MASK, trash)
        out_ref[it, :] = idx


def _knn_call(n, k, npad, br, vp):
    nb = npad // br
    return pl.pallas_call(
        functools.partial(_knn_body, n, k, npad, br),
        grid=(nb,),
        in_specs=[
            pl.BlockSpec((npad, 4), lambda i: (0, 0)),
            pl.BlockSpec((br, 4), lambda i: (i, 0)),
        ],
        out_specs=pl.BlockSpec((16, br), lambda i: (0, i)),
        out_shape=jax.ShapeDtypeStruct((16, npad), jnp.int32),
    )(vp, vp)


def _sc_scatter(npad, k, knn9r, knn_rm, xyzt3):
    mesh = plsc.VectorSubcoreMesh(core_axis_name="c", subcore_axis_name="s")
    rpt = npad // _NTILES
    cp = pltpu.CompilerParams()
    if "needs_layout_passes" in pltpu.CompilerParams.__dataclass_fields__:
        cp = dataclasses.replace(cp, needs_layout_passes=False)

    @functools.partial(
        pl.kernel,
        mesh=mesh,
        compiler_params=cp,
        out_type=jax.ShapeDtypeStruct((_NTILES, 4, 1, npad), jnp.float32),
        scratch_types=[
            pltpu.VMEM((npad,), jnp.float32),
            pltpu.VMEM((npad,), jnp.float32),
            pltpu.VMEM((npad,), jnp.float32),
            pltpu.VMEM((npad,), jnp.float32),
            pltpu.VMEM((npad,), jnp.float32),
            pltpu.VMEM((npad,), jnp.float32),
            pltpu.VMEM((npad,), jnp.float32),
            pltpu.VMEM((k, rpt), jnp.int32),
            pltpu.VMEM((rpt, 16), jnp.int32),
            pltpu.SemaphoreType.DMA,
        ],
    )
    def sck(knn9r_hbm, knnrm_hbm, xyzt_hbm, out_hbm,
            xv, yv, zv, sx, sy, sz, dg, ktb, krb, sem):
        wid = lax.axis_index("c") * 16 + lax.axis_index("s")
        base = wid * rpt

        zero16 = jnp.zeros((_L,), jnp.float32)

        @pl.loop(0, npad // _L)
        def _(i):
            sl = pl.ds(i * _L, _L)
            sx[sl] = zero16
            sy[sl] = zero16
            sz[sl] = zero16
            dg[sl] = zero16

        pltpu.sync_copy(xyzt_hbm.at[0, 0], xv)
        pltpu.sync_copy(xyzt_hbm.at[1, 0], yv)
        pltpu.sync_copy(xyzt_hbm.at[2, 0], zv)
        pltpu.sync_copy(knn9r_hbm.at[wid], ktb)
        pltpu.sync_copy(knnrm_hbm.at[pl.ds(base, rpt)], krb)

        @pl.loop(0, rpt // _L)
        def _(g):
            sl = pl.ds(g * _L, _L)
            accx = jnp.zeros((_L,), jnp.float32)
            accy = jnp.zeros((_L,), jnp.float32)
            accz = jnp.zeros((_L,), jnp.float32)
            for kk in range(k):
                u = ktb[kk, sl]
                accx += plsc.load_gather(xv, [u])
                accy += plsc.load_gather(yv, [u])
                accz += plsc.load_gather(zv, [u])
            gsl = pl.ds(base + g * _L, _L)
            sx[gsl] = sx[gsl] + accx
            sy[gsl] = sy[gsl] + accy
            sz[gsl] = sz[gsl] + accz

        ones16 = jnp.ones((_L,), jnp.float32)
        mask9 = lax.iota(jnp.int32, _L) < k
        zero16i = jnp.zeros((_L,), jnp.int32)

        @pl.loop(0, rpt)
        def _(r):
            u = krb[r]
            rg = zero16i + (base + r)
            vx = plsc.load_gather(xv, [rg])
            vy = plsc.load_gather(yv, [rg])
            vz = plsc.load_gather(zv, [rg])
            plsc.addupdate_scatter(sx, [u], vx, mask=mask9)
            plsc.addupdate_scatter(sy, [u], vy, mask=mask9)
            plsc.addupdate_scatter(sz, [u], vz, mask=mask9)
            plsc.addupdate_scatter(dg, [u], ones16, mask=mask9)

        pltpu.sync_copy(sx, out_hbm.at[wid, 0, 0])
        pltpu.sync_copy(sy, out_hbm.at[wid, 1, 0])
        pltpu.sync_copy(sz, out_hbm.at[wid, 2, 0])
        pltpu.sync_copy(dg, out_hbm.at[wid, 3, 0])

    return sck(knn9r, knn_rm, xyzt3)


def _finish_body(n, k, npad, parts_ref, xyzt_ref, out_ref):
    acc = parts_ref[0]
    for t in range(1, _NTILES):
        acc = acc + parts_ref[t]
    deg = acc[3:4, :] + jnp.float32(k)
    lx = acc[0:1, :] / deg - xyzt_ref[0:1, :]
    ly = acc[1:2, :] / deg - xyzt_ref[1:2, :]
    lz = acc[2:3, :] / deg - xyzt_ref[2:3, :]
    nrm = jnp.sqrt(lx * lx + ly * ly + lz * lz)
    valid = lax.broadcasted_iota(jnp.int32, (1, npad), 1) < n
    loss = jnp.sum(jnp.where(valid, nrm, 0.0)) / n
    out_ref[...] = loss[None, None]


def _finish_call(n, k, npad, parts, xyzt):
    return pl.pallas_call(
        functools.partial(_finish_body, n, k, npad),
        out_shape=jax.ShapeDtypeStruct((1, 1), jnp.float32),
    )(parts, xyzt)


def kernel(xyz_canon):
    vp = jnp.full((_NP, 4), _PAD_COORD, jnp.float32)
    vp = vp.at[:, 3].set(0.0).at[:_N, :3].set(xyz_canon)
    xyzt = jnp.full((3, _NP), _PAD_COORD, jnp.float32)
    xyzt = xyzt.at[:, :_N].set(xyz_canon.T)

    knnt = _knn_call(_N, _K, _NP, _BR, vp)
    knn9 = knnt[1:1 + _K]
    knn9r = knn9.reshape(_K, _NTILES, _RPT).transpose(1, 0, 2)
    knn_rm = jnp.pad(knn9.T, ((0, 0), (0, 16 - _K)))
    parts = _sc_scatter(_NP, _K, knn9r, knn_rm, xyzt.reshape(3, 1, _NP))
    loss = _finish_call(_N, _K, _NP, parts.reshape(_NTILES, 4, _NP), xyzt)
    return loss[0, 0]


# speedup vs baseline: 3.8884x; 1.2902x over previous
import dataclasses
import functools

import jax
import jax.numpy as jnp
from jax import lax
from jax.experimental import pallas as pl
from jax.experimental.pallas import tpu as pltpu
from jax.experimental.pallas import tpu_sc as plsc

_N = 10000
_K = 9
_NP = 10240
_BR = 256
_PAD_COORD = 1000.0
_REMOVED = 0x7F000000
_IDX_MASK = 16383
_NTILES = 32
_RPT = _NP // _NTILES
_L = 16


def _knn_body(n, k, npad, br, vfull_ref, vrow_ref, out_ref):
    step = pl.program_id(0)
    vp = vfull_ref[...]
    vr = vrow_ref[...]
    sq_all = jnp.sum(vp * vp, axis=1)
    sq_r = jnp.sum(vr * vr, axis=1)
    w = jnp.concatenate(
        [-2.0 * vp[:, 0:3], jnp.ones((npad, 1), jnp.float32),
         sq_all[:, None]], axis=1)
    u = jnp.concatenate(
        [vr[:, 0:3], sq_r[:, None],
         jnp.ones((br, 1), jnp.float32)], axis=1)
    d2 = lax.dot_general(
        u, w, (((1,), (1,)), ((), ())),
        preferred_element_type=jnp.float32,
        precision=lax.Precision.HIGHEST)
    cols = lax.broadcasted_iota(jnp.int32, (br, npad), 1)
    key = lax.bitcast_convert_type(
        (lax.bitcast_convert_type(d2, jnp.int32) & ~_IDX_MASK) | cols,
        jnp.float32)

    row0 = step * br
    lvalid = (row0 + lax.iota(jnp.int32, br)) < n
    trash = jnp.full((br,), npad - 1, jnp.int32)
    inf = jnp.float32(jnp.inf)
    m = jnp.min(key, axis=1)
    for it in range(1, k + 1):
        m = jnp.min(jnp.where(key > m[:, None], key, inf), axis=1)
        mi = lax.bitcast_convert_type(m, jnp.int32)
        idx = jnp.where(lvalid, mi & _IDX_MASK, trash)
        out_ref[it, :] = idx


def _knn_call(n, k, npad, br, vp):
    nb = npad // br
    return pl.pallas_call(
        functools.partial(_knn_body, n, k, npad, br),
        grid=(nb,),
        in_specs=[
            pl.BlockSpec((npad, 4), lambda i: (0, 0)),
            pl.BlockSpec((br, 4), lambda i: (i, 0)),
        ],
        out_specs=pl.BlockSpec((16, br), lambda i: (0, i)),
        out_shape=jax.ShapeDtypeStruct((16, npad), jnp.int32),
    )(vp, vp)


def _sc_scatter(npad, k, knn9r, knn_rm, xyzt3):
    mesh = plsc.VectorSubcoreMesh(core_axis_name="c", subcore_axis_name="s")
    rpt = npad // _NTILES
    cp = pltpu.CompilerParams()
    if "needs_layout_passes" in pltpu.CompilerParams.__dataclass_fields__:
        cp = dataclasses.replace(cp, needs_layout_passes=False)

    @functools.partial(
        pl.kernel,
        mesh=mesh,
        compiler_params=cp,
        out_type=jax.ShapeDtypeStruct((_NTILES, 4, 1, npad), jnp.float32),
        scratch_types=[
            pltpu.VMEM((npad,), jnp.float32),
            pltpu.VMEM((npad,), jnp.float32),
            pltpu.VMEM((npad,), jnp.float32),
            pltpu.VMEM((npad,), jnp.float32),
            pltpu.VMEM((npad,), jnp.float32),
            pltpu.VMEM((npad,), jnp.float32),
            pltpu.VMEM((npad,), jnp.float32),
            pltpu.VMEM((k, rpt), jnp.int32),
            pltpu.VMEM((rpt, 16), jnp.int32),
            pltpu.SemaphoreType.DMA,
        ],
    )
    def sck(knn9r_hbm, knnrm_hbm, xyzt_hbm, out_hbm,
            xv, yv, zv, sx, sy, sz, dg, ktb, krb, sem):
        wid = lax.axis_index("c") * 16 + lax.axis_index("s")
        base = wid * rpt

        zero16 = jnp.zeros((_L,), jnp.float32)

        @pl.loop(0, npad // _L)
        def _(i):
            sl = pl.ds(i * _L, _L)
            sx[sl] = zero16
            sy[sl] = zero16
            sz[sl] = zero16
            dg[sl] = zero16

        pltpu.sync_copy(xyzt_hbm.at[0, 0], xv)
        pltpu.sync_copy(xyzt_hbm.at[1, 0], yv)
        pltpu.sync_copy(xyzt_hbm.at[2, 0], zv)
        pltpu.sync_copy(knn9r_hbm.at[wid], ktb)
        pltpu.sync_copy(knnrm_hbm.at[pl.ds(base, rpt)], krb)

        @pl.loop(0, rpt // _L)
        def _(g):
            sl = pl.ds(g * _L, _L)
            accx = jnp.zeros((_L,), jnp.float32)
            accy = jnp.zeros((_L,), jnp.float32)
            accz = jnp.zeros((_L,), jnp.float32)
            for kk in range(k):
                u = ktb[kk, sl]
                accx += plsc.load_gather(xv, [u])
                accy += plsc.load_gather(yv, [u])
                accz += plsc.load_gather(zv, [u])
            gsl = pl.ds(base + g * _L, _L)
            sx[gsl] = sx[gsl] + accx
            sy[gsl] = sy[gsl] + accy
            sz[gsl] = sz[gsl] + accz

        ones16 = jnp.ones((_L,), jnp.float32)
        mask9 = lax.iota(jnp.int32, _L) < k
        zero16i = jnp.zeros((_L,), jnp.int32)

        @pl.loop(0, rpt)
        def _(r):
            u = krb[r]
            rg = zero16i + (base + r)
            vx = plsc.load_gather(xv, [rg])
            vy = plsc.load_gather(yv, [rg])
            vz = plsc.load_gather(zv, [rg])
            plsc.addupdate_scatter(sx, [u], vx, mask=mask9)
            plsc.addupdate_scatter(sy, [u], vy, mask=mask9)
            plsc.addupdate_scatter(sz, [u], vz, mask=mask9)
            plsc.addupdate_scatter(dg, [u], ones16, mask=mask9)

        pltpu.sync_copy(sx, out_hbm.at[wid, 0, 0])
        pltpu.sync_copy(sy, out_hbm.at[wid, 1, 0])
        pltpu.sync_copy(sz, out_hbm.at[wid, 2, 0])
        pltpu.sync_copy(dg, out_hbm.at[wid, 3, 0])

    return sck(knn9r, knn_rm, xyzt3)


def _finish_body(n, k, npad, parts_ref, xyzt_ref, out_ref):
    acc = parts_ref[0]
    for t in range(1, _NTILES):
        acc = acc + parts_ref[t]
    deg = acc[3:4, :] + jnp.float32(k)
    lx = acc[0:1, :] / deg - xyzt_ref[0:1, :]
    ly = acc[1:2, :] / deg - xyzt_ref[1:2, :]
    lz = acc[2:3, :] / deg - xyzt_ref[2:3, :]
    nrm = jnp.sqrt(lx * lx + ly * ly + lz * lz)
    valid = lax.broadcasted_iota(jnp.int32, (1, npad), 1) < n
    loss = jnp.sum(jnp.where(valid, nrm, 0.0)) / n
    out_ref[...] = loss[None, None]


def _finish_call(n, k, npad, parts, xyzt):
    return pl.pallas_call(
        functools.partial(_finish_body, n, k, npad),
        out_shape=jax.ShapeDtypeStruct((1, 1), jnp.float32),
    )(parts, xyzt)


def kernel(xyz_canon):
    vp = jnp.full((_NP, 4), _PAD_COORD, jnp.float32)
    vp = vp.at[:, 3].set(0.0).at[:_N, :3].set(xyz_canon)
    xyzt = jnp.full((3, _NP), _PAD_COORD, jnp.float32)
    xyzt = xyzt.at[:, :_N].set(xyz_canon.T)

    knnt = _knn_call(_N, _K, _NP, _BR, vp)
    knn9 = knnt[1:1 + _K]
    knn9r = knn9.reshape(_K, _NTILES, _RPT).transpose(1, 0, 2)
    knn_rm = jnp.pad(knn9.T, ((0, 0), (0, 16 - _K)))
    parts = _sc_scatter(_NP, _K, knn9r, knn_rm, xyzt.reshape(3, 1, _NP))
    loss = _finish_call(_N, _K, _NP, parts.reshape(_NTILES, 4, _NP), xyzt)
    return loss[0, 0]


# speedup vs baseline: 4.5610x; 1.1730x over previous
import dataclasses
import functools

import jax
import jax.numpy as jnp
from jax import lax
from jax.experimental import pallas as pl
from jax.experimental.pallas import tpu as pltpu
from jax.experimental.pallas import tpu_sc as plsc

_N = 10000
_K = 9
_NP = 10240
_BR = 128
_PAD_COORD = 1000.0
_REMOVED = 0x7F000000
_IDX_MASK = 16383
_NTILES = 32
_RPT = _NP // _NTILES
_L = 16


def _knn_body(n, k, npad, br, vfull_ref, vrow_ref, out_ref, key_scr):
    step = pl.program_id(0)
    vp = vfull_ref[...]
    vr = vrow_ref[...]
    sq_all = jnp.sum(vp * vp, axis=1)
    sq_r = jnp.sum(vr * vr, axis=1)
    dot = lax.dot_general(
        vr, vp, (((1,), (1,)), ((), ())),
        preferred_element_type=jnp.float32)
    d2 = sq_r[:, None] + sq_all[None, :] - 2.0 * dot
    cols = lax.broadcasted_iota(jnp.int32, (br, npad), 1)
    key_scr[...] = lax.bitcast_convert_type(
        (lax.bitcast_convert_type(d2, jnp.int32) & ~_IDX_MASK) | cols,
        jnp.float32)

    row0 = step * br
    lvalid = (row0 + lax.iota(jnp.int32, br)) < n
    trash = jnp.full((br,), npad - 1, jnp.int32)
    inf = jnp.float32(jnp.inf)
    for it in range(k + 1):
        kk = key_scr[...]
        m = jnp.min(kk, axis=1)
        if it > 0:
            mi = lax.bitcast_convert_type(m, jnp.int32)
            idx = jnp.where(lvalid, mi & _IDX_MASK, trash)
            out_ref[it, :] = idx
        if it < k:
            key_scr[...] = jnp.where(kk == m[:, None], inf, kk)


def _knn_call(n, k, npad, br, vp):
    nb = npad // br
    return pl.pallas_call(
        functools.partial(_knn_body, n, k, npad, br),
        grid=(nb,),
        in_specs=[
            pl.BlockSpec((npad, 4), lambda i: (0, 0)),
            pl.BlockSpec((br, 4), lambda i: (i, 0)),
        ],
        out_specs=pl.BlockSpec((16, br), lambda i: (0, i)),
        out_shape=jax.ShapeDtypeStruct((16, npad), jnp.int32),
        scratch_shapes=[pltpu.VMEM((br, npad), jnp.float32)],
    )(vp, vp)


def _sc_scatter(npad, k, knn9r, knn_rm, xyzt3):
    mesh = plsc.VectorSubcoreMesh(core_axis_name="c", subcore_axis_name="s")
    rpt = npad // _NTILES
    cp = pltpu.CompilerParams()
    if "needs_layout_passes" in pltpu.CompilerParams.__dataclass_fields__:
        cp = dataclasses.replace(cp, needs_layout_passes=False)

    @functools.partial(
        pl.kernel,
        mesh=mesh,
        compiler_params=cp,
        out_type=jax.ShapeDtypeStruct((_NTILES, 4, 1, npad), jnp.float32),
        scratch_types=[
            pltpu.VMEM((npad,), jnp.float32),
            pltpu.VMEM((npad,), jnp.float32),
            pltpu.VMEM((npad,), jnp.float32),
            pltpu.VMEM((npad,), jnp.float32),
            pltpu.VMEM((npad,), jnp.float32),
            pltpu.VMEM((npad,), jnp.float32),
            pltpu.VMEM((npad,), jnp.float32),
            pltpu.VMEM((k, rpt), jnp.int32),
            pltpu.VMEM((rpt, 16), jnp.int32),
            pltpu.SemaphoreType.DMA,
        ],
    )
    def sck(knn9r_hbm, knnrm_hbm, xyzt_hbm, out_hbm,
            xv, yv, zv, sx, sy, sz, dg, ktb, krb, sem):
        wid = lax.axis_index("c") * 16 + lax.axis_index("s")
        base = wid * rpt

        zero16 = jnp.zeros((_L,), jnp.float32)

        @pl.loop(0, npad // _L)
        def _(i):
            sl = pl.ds(i * _L, _L)
            sx[sl] = zero16
            sy[sl] = zero16
            sz[sl] = zero16
            dg[sl] = zero16

        pltpu.sync_copy(xyzt_hbm.at[0, 0], xv)
        pltpu.sync_copy(xyzt_hbm.at[1, 0], yv)
        pltpu.sync_copy(xyzt_hbm.at[2, 0], zv)
        pltpu.sync_copy(knn9r_hbm.at[wid], ktb)
        pltpu.sync_copy(knnrm_hbm.at[pl.ds(base, rpt)], krb)

        @pl.loop(0, rpt // _L)
        def _(g):
            sl = pl.ds(g * _L, _L)
            accx = jnp.zeros((_L,), jnp.float32)
            accy = jnp.zeros((_L,), jnp.float32)
            accz = jnp.zeros((_L,), jnp.float32)
            for kk in range(k):
                u = ktb[kk, sl]
                accx += plsc.load_gather(xv, [u])
                accy += plsc.load_gather(yv, [u])
                accz += plsc.load_gather(zv, [u])
            gsl = pl.ds(base + g * _L, _L)
            sx[gsl] = sx[gsl] + accx
            sy[gsl] = sy[gsl] + accy
            sz[gsl] = sz[gsl] + accz

        ones16 = jnp.ones((_L,), jnp.float32)
        mask9 = lax.iota(jnp.int32, _L) < k
        zero16i = jnp.zeros((_L,), jnp.int32)

        @pl.loop(0, rpt)
        def _(r):
            u = krb[r]
            rg = zero16i + (base + r)
            vx = plsc.load_gather(xv, [rg])
            vy = plsc.load_gather(yv, [rg])
            vz = plsc.load_gather(zv, [rg])
            plsc.addupdate_scatter(sx, [u], vx, mask=mask9)
            plsc.addupdate_scatter(sy, [u], vy, mask=mask9)
            plsc.addupdate_scatter(sz, [u], vz, mask=mask9)
            plsc.addupdate_scatter(dg, [u], ones16, mask=mask9)

        pltpu.sync_copy(sx, out_hbm.at[wid, 0, 0])
        pltpu.sync_copy(sy, out_hbm.at[wid, 1, 0])
        pltpu.sync_copy(sz, out_hbm.at[wid, 2, 0])
        pltpu.sync_copy(dg, out_hbm.at[wid, 3, 0])

    return sck(knn9r, knn_rm, xyzt3)


def _finish_body(n, k, npad, parts_ref, xyzt_ref, out_ref):
    acc = parts_ref[0]
    for t in range(1, _NTILES):
        acc = acc + parts_ref[t]
    deg = acc[3:4, :] + jnp.float32(k)
    lx = acc[0:1, :] / deg - xyzt_ref[0:1, :]
    ly = acc[1:2, :] / deg - xyzt_ref[1:2, :]
    lz = acc[2:3, :] / deg - xyzt_ref[2:3, :]
    nrm = jnp.sqrt(lx * lx + ly * ly + lz * lz)
    valid = lax.broadcasted_iota(jnp.int32, (1, npad), 1) < n
    loss = jnp.sum(jnp.where(valid, nrm, 0.0)) / n
    out_ref[...] = loss[None, None]


def _finish_call(n, k, npad, parts, xyzt):
    return pl.pallas_call(
        functools.partial(_finish_body, n, k, npad),
        out_shape=jax.ShapeDtypeStruct((1, 1), jnp.float32),
    )(parts, xyzt)


def kernel(xyz_canon):
    vp = jnp.full((_NP, 4), _PAD_COORD, jnp.float32)
    vp = vp.at[:, 3].set(0.0).at[:_N, :3].set(xyz_canon)
    xyzt = jnp.full((3, _NP), _PAD_COORD, jnp.float32)
    xyzt = xyzt.at[:, :_N].set(xyz_canon.T)

    knnt = _knn_call(_N, _K, _NP, _BR, vp)
    knn9 = knnt[1:1 + _K]
    knn9r = knn9.reshape(_K, _NTILES, _RPT).transpose(1, 0, 2)
    knn_rm = jnp.pad(knn9.T, ((0, 0), (0, 16 - _K)))
    parts = _sc_scatter(_NP, _K, knn9r, knn_rm, xyzt.reshape(3, 1, _NP))
    loss = _finish_call(_N, _K, _NP, parts.reshape(_NTILES, 4, _NP), xyzt)
    return loss[0, 0]


# speedup vs baseline: 4.9185x; 1.0784x over previous
import dataclasses
import functools

import jax
import jax.numpy as jnp
from jax import lax
from jax.experimental import pallas as pl
from jax.experimental.pallas import tpu as pltpu
from jax.experimental.pallas import tpu_sc as plsc

_N = 10000
_K = 9
_NP = 10240
_BR = 256
_PAD_COORD = 1000.0
_REMOVED = 0x7F000000
_IDX_MASK = 16383
_NTILES = 32
_RPT = _NP // _NTILES
_L = 16


def _knn_body(n, k, npad, br, vfull_ref, vrow_ref, out_ref, key_scr):
    step = pl.program_id(0)
    vp = vfull_ref[...]
    vr = vrow_ref[...]
    sq_all = jnp.sum(vp * vp, axis=1)
    sq_r = jnp.sum(vr * vr, axis=1)
    dot = lax.dot_general(
        vr, vp, (((1,), (1,)), ((), ())),
        preferred_element_type=jnp.float32)
    d2 = sq_r[:, None] + sq_all[None, :] - 2.0 * dot
    cols = lax.broadcasted_iota(jnp.int32, (br, npad), 1)
    key_scr[...] = lax.bitcast_convert_type(
        (lax.bitcast_convert_type(d2, jnp.int32) & ~_IDX_MASK) | cols,
        jnp.float32)

    row0 = step * br
    lvalid = (row0 + lax.iota(jnp.int32, br)) < n
    trash = jnp.full((br,), npad - 1, jnp.int32)
    inf = jnp.float32(jnp.inf)
    for it in range(k + 1):
        kk = key_scr[...]
        m = jnp.min(kk, axis=1)
        if it > 0:
            mi = lax.bitcast_convert_type(m, jnp.int32)
            idx = jnp.where(lvalid, mi & _IDX_MASK, trash)
            out_ref[it, :] = idx
        if it < k:
            key_scr[...] = jnp.where(kk == m[:, None], inf, kk)


def _knn_call(n, k, npad, br, vp):
    nb = npad // br
    return pl.pallas_call(
        functools.partial(_knn_body, n, k, npad, br),
        grid=(nb,),
        in_specs=[
            pl.BlockSpec((npad, 4), lambda i: (0, 0)),
            pl.BlockSpec((br, 4), lambda i: (i, 0)),
        ],
        out_specs=pl.BlockSpec((16, br), lambda i: (0, i)),
        out_shape=jax.ShapeDtypeStruct((16, npad), jnp.int32),
        scratch_shapes=[pltpu.VMEM((br, npad), jnp.float32)],
    )(vp, vp)


def _sc_scatter(npad, k, knn9r, knn_rm, xyzt3):
    mesh = plsc.VectorSubcoreMesh(core_axis_name="c", subcore_axis_name="s")
    rpt = npad // _NTILES
    cp = pltpu.CompilerParams()
    if "needs_layout_passes" in pltpu.CompilerParams.__dataclass_fields__:
        cp = dataclasses.replace(cp, needs_layout_passes=False)

    @functools.partial(
        pl.kernel,
        mesh=mesh,
        compiler_params=cp,
        out_type=jax.ShapeDtypeStruct((_NTILES, 4, 1, npad), jnp.float32),
        scratch_types=[
            pltpu.VMEM((npad,), jnp.float32),
            pltpu.VMEM((npad,), jnp.float32),
            pltpu.VMEM((npad,), jnp.float32),
            pltpu.VMEM((npad,), jnp.float32),
            pltpu.VMEM((npad,), jnp.float32),
            pltpu.VMEM((npad,), jnp.float32),
            pltpu.VMEM((npad,), jnp.float32),
            pltpu.VMEM((k, rpt), jnp.int32),
            pltpu.VMEM((rpt, 16), jnp.int32),
            pltpu.SemaphoreType.DMA,
        ],
    )
    def sck(knn9r_hbm, knnrm_hbm, xyzt_hbm, out_hbm,
            xv, yv, zv, sx, sy, sz, dg, ktb, krb, sem):
        wid = lax.axis_index("c") * 16 + lax.axis_index("s")
        base = wid * rpt

        zero16 = jnp.zeros((_L,), jnp.float32)

        @pl.loop(0, npad // _L)
        def _(i):
            sl = pl.ds(i * _L, _L)
            sx[sl] = zero16
            sy[sl] = zero16
            sz[sl] = zero16
            dg[sl] = zero16

        pltpu.sync_copy(xyzt_hbm.at[0, 0], xv)
        pltpu.sync_copy(xyzt_hbm.at[1, 0], yv)
        pltpu.sync_copy(xyzt_hbm.at[2, 0], zv)
        pltpu.sync_copy(knn9r_hbm.at[wid], ktb)
        pltpu.sync_copy(knnrm_hbm.at[pl.ds(base, rpt)], krb)

        @pl.loop(0, rpt // _L)
        def _(g):
            sl = pl.ds(g * _L, _L)
            accx = jnp.zeros((_L,), jnp.float32)
            accy = jnp.zeros((_L,), jnp.float32)
            accz = jnp.zeros((_L,), jnp.float32)
            for kk in range(k):
                u = ktb[kk, sl]
                accx += plsc.load_gather(xv, [u])
                accy += plsc.load_gather(yv, [u])
                accz += plsc.load_gather(zv, [u])
            gsl = pl.ds(base + g * _L, _L)
            sx[gsl] = sx[gsl] + accx
            sy[gsl] = sy[gsl] + accy
            sz[gsl] = sz[gsl] + accz

        ones16 = jnp.ones((_L,), jnp.float32)
        mask9 = lax.iota(jnp.int32, _L) < k
        zero16i = jnp.zeros((_L,), jnp.int32)

        @pl.loop(0, rpt)
        def _(r):
            u = krb[r]
            rg = zero16i + (base + r)
            vx = plsc.load_gather(xv, [rg])
            vy = plsc.load_gather(yv, [rg])
            vz = plsc.load_gather(zv, [rg])
            plsc.addupdate_scatter(sx, [u], vx, mask=mask9)
            plsc.addupdate_scatter(sy, [u], vy, mask=mask9)
            plsc.addupdate_scatter(sz, [u], vz, mask=mask9)
            plsc.addupdate_scatter(dg, [u], ones16, mask=mask9)

        pltpu.sync_copy(sx, out_hbm.at[wid, 0, 0])
        pltpu.sync_copy(sy, out_hbm.at[wid, 1, 0])
        pltpu.sync_copy(sz, out_hbm.at[wid, 2, 0])
        pltpu.sync_copy(dg, out_hbm.at[wid, 3, 0])

    return sck(knn9r, knn_rm, xyzt3)


def _finish_body(n, k, npad, parts_ref, xyzt_ref, out_ref):
    acc = parts_ref[0]
    for t in range(1, _NTILES):
        acc = acc + parts_ref[t]
    deg = acc[3:4, :] + jnp.float32(k)
    lx = acc[0:1, :] / deg - xyzt_ref[0:1, :]
    ly = acc[1:2, :] / deg - xyzt_ref[1:2, :]
    lz = acc[2:3, :] / deg - xyzt_ref[2:3, :]
    nrm = jnp.sqrt(lx * lx + ly * ly + lz * lz)
    valid = lax.broadcasted_iota(jnp.int32, (1, npad), 1) < n
    loss = jnp.sum(jnp.where(valid, nrm, 0.0)) / n
    out_ref[...] = loss[None, None]


def _finish_call(n, k, npad, parts, xyzt):
    return pl.pallas_call(
        functools.partial(_finish_body, n, k, npad),
        out_shape=jax.ShapeDtypeStruct((1, 1), jnp.float32),
    )(parts, xyzt)


def kernel(xyz_canon):
    vp = jnp.full((_NP, 4), _PAD_COORD, jnp.float32)
    vp = vp.at[:, 3].set(0.0).at[:_N, :3].set(xyz_canon)
    xyzt = jnp.full((3, _NP), _PAD_COORD, jnp.float32)
    xyzt = xyzt.at[:, :_N].set(xyz_canon.T)

    knnt = _knn_call(_N, _K, _NP, _BR, vp)
    knn9 = knnt[1:1 + _K]
    knn9r = knn9.reshape(_K, _NTILES, _RPT).transpose(1, 0, 2)
    knn_rm = jnp.pad(knn9.T, ((0, 0), (0, 16 - _K)))
    parts = _sc_scatter(_NP, _K, knn9r, knn_rm, xyzt.reshape(3, 1, _NP))
    loss = _finish_call(_N, _K, _NP, parts.reshape(_NTILES, 4, _NP), xyzt)
    return loss[0, 0]
